# baseline (speedup 1.0000x reference)
import functools

import jax
import jax.numpy as jnp
from jax import lax
from jax.experimental import pallas as pl
from jax.experimental.pallas import tpu as pltpu
from jax.experimental.pallas import tpu_sc as plsc

B, S, D = 1, 2048, 768
E, K, DFF = 8, 2, 1024
T = B * S
NSLOT = K * T
BM = 128
NB_E0 = T // BM
NB_R = NSLOT // BM + (E - 2)
NR = NB_R * BM
G = NB_E0 + NB_R
NROWS_Y = T + NR
LANES = 16


def _plan_kernel(x_ref, wr_ref, posw_ref, be_ref):
    x = x_ref[...]
    lane8 = lax.broadcasted_iota(jnp.int32, (T, E), 1)
    l = jnp.dot(x, wr_ref[...], preferred_element_type=jnp.float32)
    l = jnp.where(lane8 < E - 1, l, -1e30)
    m1 = jnp.max(l, axis=1, keepdims=True)
    idx1 = jnp.min(jnp.where(l == m1, lane8, E + 9), axis=1, keepdims=True)
    l2 = jnp.where(lane8 == idx1, -1e30, l)
    m2 = jnp.max(l2, axis=1, keepdims=True)
    idx2 = jnp.min(jnp.where(l2 == m2, lane8, E + 9), axis=1, keepdims=True)
    p2 = jnp.exp(m2 - m1)
    w1n = 1.0 / (1.0 + p2)
    w2n = p2 / (1.0 + p2)

    lanef = lane8.astype(jnp.float32)
    tcols = (
        idx1.astype(jnp.float32) * (lanef == 0.0)
        + idx2.astype(jnp.float32) * (lanef == 1.0)
        + w1n * (lanef == 2.0)
        + w2n * (lanef == 3.0)
    )
    tt = jnp.transpose(tcols)
    idx1L = tt[0:1, :]
    idx2L = tt[1:2, :]
    w1L = tt[2:3, :]
    w2L = tt[3:4, :]

    subi = lax.broadcasted_iota(jnp.int32, (E, T), 0).astype(jnp.float32)
    oh1 = (subi == idx1L).astype(jnp.float32)
    oh2 = (subi == idx2L).astype(jnp.float32)

    upper = (
        lax.broadcasted_iota(jnp.int32, (BM, BM), 0)
        < lax.broadcasted_iota(jnp.int32, (BM, BM), 1)
    ).astype(jnp.float32)
    hi = jax.lax.Precision.HIGHEST

    def ex_cumsum(oh):
        segs = []
        carry = jnp.zeros((E, 1), jnp.float32)
        for ci in range(T // BM):
            seg = oh[:, ci * BM : (ci + 1) * BM]
            segs.append(
                jax.lax.dot(seg, upper, precision=hi,
                            preferred_element_type=jnp.float32) + carry)
            carry = carry + jnp.sum(seg, axis=1, keepdims=True)
        return jnp.concatenate(segs, axis=1), carry

    c1, tot1 = ex_cumsum(oh1)
    c2, _ = ex_cumsum(oh2)
    cnt = tot1 + jnp.sum(oh2, axis=1, keepdims=True)
    pc = jnp.floor((cnt + (BM - 1)) * (1.0 / BM)) * BM
    ltri = (
        lax.broadcasted_iota(jnp.int32, (E, E), 0)
        > lax.broadcasted_iota(jnp.int32, (E, E), 1)
    ).astype(jnp.float32)
    off = jax.lax.dot(ltri, pc, precision=hi,
                      preferred_element_type=jnp.float32)
    offy = off + float(T)

    pos1 = jnp.sum(oh1 * (c1 + offy), axis=0, keepdims=True)
    pos2 = jnp.sum(oh2 * (c2 + tot1 + offy), axis=0, keepdims=True)
    posw_ref[...] = jnp.concatenate([pos1, pos2, w1L, w2L], axis=0)

    jlane = lax.broadcasted_iota(jnp.int32, (E, 64), 1).astype(jnp.float32)
    bstart = (jlane - float(NB_E0)) * float(BM)
    rowok = lax.broadcasted_iota(jnp.int32, (E, 64), 0) < (E - 1)
    ind = jnp.where((bstart >= off) & rowok, 1.0, 0.0)
    be_ref[...] = jnp.sum(ind, axis=0, keepdims=True).astype(jnp.int32)


def _grouped_kernel(be_ref, x16_ref, xp_ref, w1_ref, w2_ref, yp_ref):
    b = pl.program_id(0)
    xb = jnp.where(b < NB_E0, x16_ref[...], xp_ref[...].astype(jnp.bfloat16))
    h = jnp.dot(xb, w1_ref[0], preferred_element_type=jnp.float32)
    h = h * jax.lax.logistic(h)
    yp_ref[...] = jnp.dot(h.astype(jnp.bfloat16), w2_ref[0],
                          preferred_element_type=jnp.float32)


def _dispatch_body(posw_hbm, x_hbm, xp_hbm, posf_v, idx_v, rows_v, sem):
    c = lax.axis_index("c")
    s = lax.axis_index("s")
    wid = s * 2 + c
    k = wid // NB_E0
    t0 = (wid % NB_E0) * BM
    pltpu.sync_copy(posw_hbm.at[pl.ds(k, 1), pl.ds(t0, BM)], posf_v)
    for ci in range(BM // LANES):
        sl = pl.ds(ci * LANES, LANES)
        idx_v[sl] = (posf_v[0, sl] - float(T)).astype(jnp.int32)
    pltpu.sync_copy(x_hbm.at[pl.ds(t0, BM)], rows_v)
    pltpu.async_copy(rows_v, xp_hbm.at[idx_v], sem).wait()


def _combine_body(yp_hbm, posw_hbm, wsexp_hbm, out_hbm,
                  acc_v, g_v, posf_v, idx_v, wexp_v, sem):
    c = lax.axis_index("c")
    s = lax.axis_index("s")
    wid = s * 2 + c
    nt = T // 32
    t0 = wid * nt
    t0a = (wid // 2) * 128
    half = (wid % 2) * nt
    pltpu.sync_copy(yp_hbm.at[pl.ds(t0, nt)], acc_v)
    for k in range(K):
        pltpu.sync_copy(posw_hbm.at[pl.ds(k, 1), pl.ds(t0a, 128)], posf_v)
        for ci in range(nt // LANES):
            idx_v[pl.ds(ci * LANES, LANES)] = (
                posf_v[0, pl.ds(half + ci * LANES, LANES)].astype(jnp.int32))
        pltpu.async_copy(yp_hbm.at[idx_v], g_v, sem).wait()
        pltpu.sync_copy(wsexp_hbm.at[pl.ds(k * T + t0, nt)], wexp_v)

        def body(j, carry):
            wv = wexp_v[j, pl.ds(0, LANES)]
            for ci in range(D // LANES):
                sl = pl.ds(ci * LANES, LANES)
                acc_v[j, sl] = acc_v[j, sl] + wv * g_v[j, sl]
            return carry

        lax.fori_loop(0, nt, body, 0)
    pltpu.sync_copy(acc_v, out_hbm.at[pl.ds(t0, nt)])


def kernel(hidden_states, Wr, W1, W2, interpret=False):
    x = hidden_states.reshape(T, D)
    wr_pad = jnp.zeros((D, E), jnp.float32).at[:, : E - 1].set(Wr)
    x16 = x.astype(jnp.bfloat16)
    w1b = W1.astype(jnp.bfloat16)
    w2b = W2.astype(jnp.bfloat16)

    posw, be2d = pl.pallas_call(
        _plan_kernel,
        grid=(1,),
        in_specs=[
            pl.BlockSpec((T, D), lambda i: (0, 0)),
            pl.BlockSpec((D, E), lambda i: (0, 0)),
        ],
        out_specs=[
            pl.BlockSpec((4, T), lambda i: (0, 0)),
            pl.BlockSpec((1, 64), lambda i: (0, 0)),
        ],
        out_shape=[
            jax.ShapeDtypeStruct((4, T), jnp.float32),
            jax.ShapeDtypeStruct((1, 64), jnp.int32),
        ],
        interpret=interpret,
    )(x, wr_pad)
    be = be2d.reshape(64)
    wsexp = jnp.repeat(posw[2:4].reshape(NSLOT, 1), 128, axis=1)

    mesh = plsc.VectorSubcoreMesh(core_axis_name="c", subcore_axis_name="s")
    xp = pl.kernel(
        _dispatch_body,
        mesh=mesh,
        out_type=jax.ShapeDtypeStruct((NR, D), jnp.float32),
        scratch_types=[
            pltpu.VMEM((1, BM), jnp.float32),
            pltpu.VMEM((BM,), jnp.int32),
            pltpu.VMEM((BM, D), jnp.float32),
            pltpu.SemaphoreType.DMA,
        ],
    )(posw, x)

    yp = pl.pallas_call(
        _grouped_kernel,
        grid_spec=pltpu.PrefetchScalarGridSpec(
            num_scalar_prefetch=1,
            grid=(G,),
            in_specs=[
                pl.BlockSpec((BM, D), lambda b, be_r: (jnp.minimum(b, NB_E0 - 1), 0)),
                pl.BlockSpec((BM, D), lambda b, be_r: (jnp.maximum(b - NB_E0, 0), 0)),
                pl.BlockSpec((1, D, DFF), lambda b, be_r: (be_r[b], 0, 0)),
                pl.BlockSpec((1, DFF, D), lambda b, be_r: (be_r[b], 0, 0)),
            ],
            out_specs=pl.BlockSpec((BM, D), lambda b, be_r: (b, 0)),
        ),
        out_shape=jax.ShapeDtypeStruct((NROWS_Y, D), jnp.float32),
        interpret=interpret,
    )(be, x16, xp, w1b, w2b)

    out = pl.kernel(
        _combine_body,
        mesh=mesh,
        out_type=jax.ShapeDtypeStruct((T, D), jnp.float32),
        scratch_types=[
            pltpu.VMEM((T // 32, D), jnp.float32),
            pltpu.VMEM((T // 32, D), jnp.float32),
            pltpu.VMEM((1, 128), jnp.float32),
            pltpu.VMEM((T // 32,), jnp.int32),
            pltpu.VMEM((T // 32, 128), jnp.float32),
            pltpu.SemaphoreType.DMA,
        ],
    )(yp, posw, wsexp)
    return out.reshape(B, S, D)


# speedup vs baseline: 1.1417x; 1.1417x over previous
import functools

import jax
import jax.numpy as jnp
from jax import lax
from jax.experimental import pallas as pl
from jax.experimental.pallas import tpu as pltpu
from jax.experimental.pallas import tpu_sc as plsc

B, S, D = 1, 2048, 768
E, K, DFF = 8, 2, 1024
T = B * S
NSLOT = K * T
BM = 128
NB_E0 = T // BM
NB_R = NSLOT // BM + (E - 2)
NR = NB_R * BM
G = NB_E0 + NB_R
NROWS_Y = T + NR
LANES = 16


def _plan_kernel(x_ref, wr_ref, posw_ref, be_ref):
    x = x_ref[...]
    lane8 = lax.broadcasted_iota(jnp.int32, (T, E), 1)
    l = jnp.dot(x, wr_ref[...], preferred_element_type=jnp.float32)
    l = jnp.where(lane8 < E - 1, l, -1e30)
    m1 = jnp.max(l, axis=1, keepdims=True)
    idx1 = jnp.min(jnp.where(l == m1, lane8, E + 9), axis=1, keepdims=True)
    l2 = jnp.where(lane8 == idx1, -1e30, l)
    m2 = jnp.max(l2, axis=1, keepdims=True)
    idx2 = jnp.min(jnp.where(l2 == m2, lane8, E + 9), axis=1, keepdims=True)
    p2 = jnp.exp(m2 - m1)
    w1n = 1.0 / (1.0 + p2)
    w2n = p2 / (1.0 + p2)

    lanef = lane8.astype(jnp.float32)
    tcols = (
        idx1.astype(jnp.float32) * (lanef == 0.0)
        + idx2.astype(jnp.float32) * (lanef == 1.0)
        + w1n * (lanef == 2.0)
        + w2n * (lanef == 3.0)
    )
    tt = jnp.transpose(tcols)
    idx1L = tt[0:1, :]
    idx2L = tt[1:2, :]
    w1L = tt[2:3, :]
    w2L = tt[3:4, :]

    subi = lax.broadcasted_iota(jnp.int32, (E, T), 0).astype(jnp.float32)
    oh1 = (subi == idx1L).astype(jnp.float32)
    oh2 = (subi == idx2L).astype(jnp.float32)

    upper = (
        lax.broadcasted_iota(jnp.int32, (BM, BM), 0)
        < lax.broadcasted_iota(jnp.int32, (BM, BM), 1)
    ).astype(jnp.float32)
    hi = jax.lax.Precision.HIGHEST

    def ex_cumsum(oh):
        segs = []
        carry = jnp.zeros((E, 1), jnp.float32)
        for ci in range(T // BM):
            seg = oh[:, ci * BM : (ci + 1) * BM]
            segs.append(
                jax.lax.dot(seg, upper, precision=hi,
                            preferred_element_type=jnp.float32) + carry)
            carry = carry + jnp.sum(seg, axis=1, keepdims=True)
        return jnp.concatenate(segs, axis=1), carry

    c1, tot1 = ex_cumsum(oh1)
    c2, _ = ex_cumsum(oh2)
    cnt = tot1 + jnp.sum(oh2, axis=1, keepdims=True)
    pc = jnp.floor((cnt + (BM - 1)) * (1.0 / BM)) * BM
    ltri = (
        lax.broadcasted_iota(jnp.int32, (E, E), 0)
        > lax.broadcasted_iota(jnp.int32, (E, E), 1)
    ).astype(jnp.float32)
    off = jax.lax.dot(ltri, pc, precision=hi,
                      preferred_element_type=jnp.float32)
    offy = off + float(T)

    pos1 = jnp.sum(oh1 * (c1 + offy), axis=0, keepdims=True)
    pos2 = jnp.sum(oh2 * (c2 + tot1 + offy), axis=0, keepdims=True)
    posw_ref[...] = jnp.concatenate([pos1, pos2, w1L, w2L], axis=0)

    jlane = lax.broadcasted_iota(jnp.int32, (E, 64), 1).astype(jnp.float32)
    bstart = (jlane - float(NB_E0)) * float(BM)
    rowok = lax.broadcasted_iota(jnp.int32, (E, 64), 0) < (E - 1)
    ind = jnp.where((bstart >= off) & rowok, 1.0, 0.0)
    be_ref[...] = jnp.sum(ind, axis=0, keepdims=True).astype(jnp.int32)


def _grouped_kernel(be_ref, x_ref, xp_ref, w1_ref, w2_ref, yp_ref):
    b = pl.program_id(0)
    xb = jnp.where(b < NB_E0, x_ref[...], xp_ref[...])
    h = jnp.dot(xb, w1_ref[0], preferred_element_type=jnp.float32)
    h = h * jax.lax.logistic(h)
    yp_ref[...] = jnp.dot(h, w2_ref[0], preferred_element_type=jnp.float32)


def _dispatch_body(posw_hbm, x_hbm, xp_hbm, posf_v, idx_v, rows_v, sem):
    c = lax.axis_index("c")
    s = lax.axis_index("s")
    wid = s * 2 + c
    k = wid // NB_E0
    t0 = (wid % NB_E0) * BM
    pltpu.sync_copy(posw_hbm.at[pl.ds(k, 1), pl.ds(t0, BM)], posf_v)
    for ci in range(BM // LANES):
        sl = pl.ds(ci * LANES, LANES)
        idx_v[sl] = (posf_v[0, sl] - float(T)).astype(jnp.int32)
    pltpu.sync_copy(x_hbm.at[pl.ds(t0, BM)], rows_v)
    pltpu.async_copy(rows_v, xp_hbm.at[idx_v], sem).wait()


def _combine_body(yp_hbm, posw_hbm, wsexp_hbm, out_hbm,
                  acc_v, g_v, posf_v, idx_v, wexp_v, sem):
    c = lax.axis_index("c")
    s = lax.axis_index("s")
    wid = s * 2 + c
    nt = T // 32
    t0 = wid * nt
    t0a = (wid // 2) * 128
    half = (wid % 2) * nt
    pltpu.sync_copy(yp_hbm.at[pl.ds(t0, nt)], acc_v)
    for k in range(K):
        pltpu.sync_copy(posw_hbm.at[pl.ds(k, 1), pl.ds(t0a, 128)], posf_v)
        for ci in range(nt // LANES):
            idx_v[pl.ds(ci * LANES, LANES)] = (
                posf_v[0, pl.ds(half + ci * LANES, LANES)].astype(jnp.int32))
        pltpu.async_copy(yp_hbm.at[idx_v], g_v, sem).wait()
        pltpu.sync_copy(wsexp_hbm.at[pl.ds(k * T + t0, nt)], wexp_v)

        def body(j, carry):
            wv = wexp_v[j, pl.ds(0, LANES)]
            for ci in range(D // LANES):
                sl = pl.ds(ci * LANES, LANES)
                acc_v[j, sl] = acc_v[j, sl] + wv * g_v[j, sl]
            return carry

        lax.fori_loop(0, nt, body, 0)
    pltpu.sync_copy(acc_v, out_hbm.at[pl.ds(t0, nt)])


def kernel(hidden_states, Wr, W1, W2, interpret=False):
    x = hidden_states.reshape(T, D)
    wr_pad = jnp.zeros((D, E), jnp.float32).at[:, : E - 1].set(Wr)

    posw, be2d = pl.pallas_call(
        _plan_kernel,
        grid=(1,),
        in_specs=[
            pl.BlockSpec((T, D), lambda i: (0, 0)),
            pl.BlockSpec((D, E), lambda i: (0, 0)),
        ],
        out_specs=[
            pl.BlockSpec((4, T), lambda i: (0, 0)),
            pl.BlockSpec((1, 64), lambda i: (0, 0)),
        ],
        out_shape=[
            jax.ShapeDtypeStruct((4, T), jnp.float32),
            jax.ShapeDtypeStruct((1, 64), jnp.int32),
        ],
        interpret=interpret,
    )(x, wr_pad)
    be = be2d.reshape(64)
    wsexp = jnp.repeat(posw[2:4].reshape(NSLOT, 1), 128, axis=1)

    mesh = plsc.VectorSubcoreMesh(core_axis_name="c", subcore_axis_name="s")
    xp = pl.kernel(
        _dispatch_body,
        mesh=mesh,
        out_type=jax.ShapeDtypeStruct((NR, D), jnp.float32),
        scratch_types=[
            pltpu.VMEM((1, BM), jnp.float32),
            pltpu.VMEM((BM,), jnp.int32),
            pltpu.VMEM((BM, D), jnp.float32),
            pltpu.SemaphoreType.DMA,
        ],
    )(posw, x)

    yp = pl.pallas_call(
        _grouped_kernel,
        grid_spec=pltpu.PrefetchScalarGridSpec(
            num_scalar_prefetch=1,
            grid=(G,),
            in_specs=[
                pl.BlockSpec((BM, D), lambda b, be_r: (jnp.minimum(b, NB_E0 - 1), 0)),
                pl.BlockSpec((BM, D), lambda b, be_r: (jnp.maximum(b - NB_E0, 0), 0)),
                pl.BlockSpec((1, D, DFF), lambda b, be_r: (be_r[b], 0, 0)),
                pl.BlockSpec((1, DFF, D), lambda b, be_r: (be_r[b], 0, 0)),
            ],
            out_specs=pl.BlockSpec((BM, D), lambda b, be_r: (b, 0)),
        ),
        out_shape=jax.ShapeDtypeStruct((NROWS_Y, D), jnp.float32),
        interpret=interpret,
    )(be, x, xp, W1, W2)

    out = pl.kernel(
        _combine_body,
        mesh=mesh,
        out_type=jax.ShapeDtypeStruct((T, D), jnp.float32),
        scratch_types=[
            pltpu.VMEM((T // 32, D), jnp.float32),
            pltpu.VMEM((T // 32, D), jnp.float32),
            pltpu.VMEM((1, 128), jnp.float32),
            pltpu.VMEM((T // 32,), jnp.int32),
            pltpu.VMEM((T // 32, 128), jnp.float32),
            pltpu.SemaphoreType.DMA,
        ],
    )(yp, posw, wsexp)
    return out.reshape(B, S, D)


# speedup vs baseline: 1.9360x; 1.6957x over previous
import functools

import jax
import jax.numpy as jnp
from jax.experimental import pallas as pl
from jax.experimental.pallas import tpu as pltpu

B, S, D = 1, 2048, 768
E, K, DFF = 8, 2, 1024
T = B * S
TB = 2048
NTB = T // TB
NDC = 1
DC = DFF // NDC


def _router_kernel(x_ref, wr_ref, w_ref):
    x = x_ref[...]
    lane = jax.lax.broadcasted_iota(jnp.int32, (TB, E), 1)
    l = jnp.dot(x, wr_ref[...], preferred_element_type=jnp.float32)
    l = jnp.where(lane < E - 1, l, -1e30)
    m1 = jnp.max(l, axis=1, keepdims=True)
    idx1 = jnp.min(jnp.where(l == m1, lane, E + 9), axis=1, keepdims=True)
    l2 = jnp.where(lane == idx1, -1e30, l)
    m2 = jnp.max(l2, axis=1, keepdims=True)
    idx2 = jnp.min(jnp.where(l2 == m2, lane, E + 9), axis=1, keepdims=True)
    p2 = jnp.exp(m2 - m1)
    denom = 1.0 + p2
    wfull = jnp.where(lane == idx1 + 1, 1.0 / denom, 0.0)
    wfull = wfull + jnp.where(lane == idx2 + 1, p2 / denom, 0.0)
    wfull = wfull + jnp.where(lane == 0, 1.0, 0.0)
    w_ref[...] = wfull


def _moe_dense_kernel(x_ref, w1_ref, w2_ref, w_ref, out_ref):
    e = pl.program_id(0)
    dc = pl.program_id(1)

    x = x_ref[...]
    h = jnp.dot(x, w1_ref[0], preferred_element_type=jnp.float32)
    h = h * jax.lax.logistic(h)
    y = jnp.dot(h, w2_ref[0], preferred_element_type=jnp.float32)

    lane = jax.lax.broadcasted_iota(jnp.int32, (TB, E), 1)
    wcol = jnp.sum(jnp.where(lane == e, w_ref[...], 0.0), axis=1, keepdims=True)
    contrib = y * wcol

    @pl.when(e == 0)
    def _init():
        out_ref[...] = contrib

    @pl.when(e > 0)
    def _acc():
        out_ref[...] += contrib


def kernel(hidden_states, Wr, W1, W2, interpret=False):
    x = hidden_states.reshape(T, D)
    wr_pad = jnp.zeros((D, E), jnp.float32).at[:, : E - 1].set(Wr)

    wfull = pl.pallas_call(
        _router_kernel,
        grid=(NTB,),
        in_specs=[
            pl.BlockSpec((TB, D), lambda tb: (tb, 0)),
            pl.BlockSpec((D, E), lambda tb: (0, 0)),
        ],
        out_specs=pl.BlockSpec((TB, E), lambda tb: (tb, 0)),
        out_shape=jax.ShapeDtypeStruct((T, E), jnp.float32),
        interpret=interpret,
    )(x, wr_pad)

    out = pl.pallas_call(
        _moe_dense_kernel,
        grid=(E, NDC),
        in_specs=[
            pl.BlockSpec((T, D), lambda e, dc: (0, 0)),
            pl.BlockSpec((1, D, DC), lambda e, dc: (e, 0, dc)),
            pl.BlockSpec((1, DC, D), lambda e, dc: (e, dc, 0)),
            pl.BlockSpec((T, E), lambda e, dc: (0, 0)),
        ],
        out_specs=pl.BlockSpec((T, D), lambda e, dc: (0, 0)),
        out_shape=jax.ShapeDtypeStruct((T, D), jnp.float32),
        interpret=interpret,
    )(x, W1, W2, wfull)
    return out.reshape(B, S, D)
